# baseline (speedup 1.0000x reference)
import functools

import jax
import jax.numpy as jnp
from jax import lax
from jax.experimental import pallas as pl
from jax.experimental.pallas import tpu as pltpu
from jax.experimental.pallas import tpu_sc as plsc

_N_FIELDS = 26
_VOCAB = 100000
_D = 16
_BATCH = 16384
_NW = 32
_NTASK = _N_FIELDS * _D
_TPW = _NTASK // _NW
_QTR = _BATCH // 4

_mesh = plsc.VectorSubcoreMesh(
    core_axis_name="c", subcore_axis_name="s", num_cores=2, num_subcores=16
)


@functools.partial(
    pl.kernel,
    out_type=jax.ShapeDtypeStruct((_N_FIELDS, _D, _BATCH), jnp.float32),
    mesh=_mesh,
    scratch_types=[
        pltpu.VMEM((_VOCAB,), jnp.float32),
        pltpu.VMEM((_BATCH,), jnp.int32),
        pltpu.VMEM((2, _QTR), jnp.float32),
        pltpu.SemaphoreType.DMA,
        pltpu.SemaphoreType.DMA,
        pltpu.SemaphoreType.DMA,
    ],
    compiler_params=pltpu.CompilerParams(
        use_tc_tiling_on_sc=True, needs_layout_passes=False
    ),
)
def _embed_gather(xt_hbm, tbl_hbm, out_hbm, row_v, idx_v, out_v, rsem, os0, os1):
    wid = lax.axis_index("s") * 2 + lax.axis_index("c")
    t0 = wid * _TPW
    osem = (os0, os1)

    def gather_chunk(q, f, d):
        b = q % 2

        @plsc.parallel_loop(0, _QTR // 16, unroll=16)
        def _(i):
            iv = idx_v[pl.ds(q * _QTR + i * 16, 16)]
            out_v[b, pl.ds(i * 16, 16)] = plsc.load_gather(row_v, [iv])

        pltpu.async_copy(
            out_v.at[b], out_hbm.at[f, d, pl.ds(q * _QTR, _QTR)], osem[b]
        )

    def out_drain(f, d, b):
        pltpu.make_async_copy(
            out_v.at[b], out_hbm.at[f, d, pl.ds(0, _QTR)], osem[b]
        ).wait()

    f = t0 // _D
    d = t0 % _D
    row_dma = pltpu.async_copy(tbl_hbm.at[f, d], row_v, rsem)
    pltpu.sync_copy(xt_hbm.at[f], idx_v)
    row_dma.wait()
    for q in range(4):
        if q >= 2:
            out_drain(f, d, q % 2)
        gather_chunk(q, f, d)

    def task_body(t, f_prev):
        f = t // _D
        d = t % _D
        row_dma = pltpu.async_copy(tbl_hbm.at[f, d], row_v, rsem)

        @pl.when(f != f_prev)
        def _():
            pltpu.sync_copy(xt_hbm.at[f], idx_v)

        row_dma.wait()
        for q in range(4):
            out_drain(f, d, q % 2)
            gather_chunk(q, f, d)
        return f

    f_last = lax.fori_loop(t0 + 1, t0 + _TPW, task_body, f)
    d_last = (t0 + _TPW - 1) % _D
    out_drain(f_last, d_last, 0)
    out_drain(f_last, d_last, 1)


def kernel(x, tables):
    xt = x.astype(jnp.int32).T
    tbl = tables.transpose(0, 2, 1)
    out = _embed_gather(xt, tbl)
    return out.transpose(2, 0, 1)
